# baseline (speedup 1.0000x reference)
import jax
import jax.numpy as jnp
from jax import lax
from jax.experimental import pallas as pl
from jax.experimental.pallas import tpu as pltpu
from jax.experimental.pallas import tpu_sc as plsc

_B = 16384
_D = 32
_NW = 32
_CHUNK = 128
_NCH = (_B // _NW) // _CHUNK


def _sc_gather_body(idx_hbm, table_hbm, out_hbm, idx_v, rows_v, sem):
    wid = lax.axis_index("s") * 2 + lax.axis_index("c")
    base = wid * _NCH
    pltpu.sync_copy(idx_hbm.at[pl.ds(base, _NCH)], idx_v)
    copies = [
        pltpu.async_copy(table_hbm.at[idx_v.at[j]], rows_v.at[j], sem)
        for j in range(_NCH)
    ]
    for cp in copies:
        cp.wait()
    pltpu.sync_copy(rows_v, out_hbm.at[pl.ds(base, _NCH)])


_sc_gather = pl.kernel(
    _sc_gather_body,
    out_type=jax.ShapeDtypeStruct((_B // _CHUNK, _CHUNK, _D), jnp.float32),
    mesh=plsc.VectorSubcoreMesh(core_axis_name="c", subcore_axis_name="s"),
    compiler_params=pltpu.CompilerParams(use_tc_tiling_on_sc=False),
    scratch_types=[
        pltpu.VMEM((_NCH, _CHUNK), jnp.int32),
        pltpu.VMEM((_NCH, _CHUNK, _D), jnp.float32),
        pltpu.SemaphoreType.DMA,
    ],
)


def _tc_body(u_ref, rows_ref, y_ref, out_ref):
    u = u_ref[...]
    rows = rows_ref[...]
    y = y_ref[...]
    x = lax.dot_general(
        u, rows, (((1,), (1,)), ((), ())),
        preferred_element_type=jnp.float32,
    )
    bce = jnp.maximum(x, 0.0) - x * y + jnp.log1p(jnp.exp(-jnp.abs(x)))
    reg = 0.01 * jnp.sqrt(jnp.sum(u * u))
    out_ref[0, 0] = jnp.sum(bce) + reg


def kernel(item, matrix, user_embeddings, item_embeddings):
    idx = item.astype(jnp.int32).reshape(_B // _CHUNK, _CHUNK)
    rows = _sc_gather(idx, item_embeddings)
    rows = rows.reshape(_B, _D)
    y2d = matrix.reshape(1, _B)
    out = pl.pallas_call(
        _tc_body,
        out_shape=jax.ShapeDtypeStruct((1, 1), jnp.float32),
        in_specs=[
            pl.BlockSpec(memory_space=pltpu.VMEM),
            pl.BlockSpec(memory_space=pltpu.VMEM),
            pl.BlockSpec(memory_space=pltpu.VMEM),
        ],
        out_specs=pl.BlockSpec(memory_space=pltpu.SMEM),
    )(user_embeddings, rows, y2d)
    return out[0, 0]
